# Initial kernel scaffold; baseline (speedup 1.0000x reference)
import jax
import jax.numpy as jnp
from jax.experimental import pallas as pl


def kernel(xyz, feature, ori_relative_feature, neighbors_idx, W1, b1, g1, be1, W_attn, W_out, b_out, g_out, be_out, W_sc, b_sc, g_sc, be_sc):
    raise NotImplementedError("write your pallas kernel here")


# speedup vs baseline: 13.5263x; 13.5263x over previous
import functools

import jax
import jax.numpy as jnp
from jax import lax
from jax.experimental import pallas as pl
from jax.experimental.pallas import tpu as pltpu
from jax.experimental.pallas import tpu_sc as plsc

B, N, K = 4, 16384, 16
CIN, CREL, COUT = 32, 32, 64
P = B * N
R = P * K
EPS = 1e-5


def _sc_gather(table, idx2d):
    info = plsc.get_sparse_core_info()
    nw = info.num_cores * info.num_subcores
    rows_w = R // nw
    idx_rows = rows_w // 128
    ch = 2048
    nch = rows_w // ch
    dmas = ch // 128
    mesh = plsc.VectorSubcoreMesh(core_axis_name="c", subcore_axis_name="s")

    @functools.partial(
        pl.kernel,
        mesh=mesh,
        compiler_params=pltpu.CompilerParams(use_tc_tiling_on_sc=False),
        out_type=jax.ShapeDtypeStruct((R, CIN), jnp.float32),
        scratch_types=[
            pltpu.VMEM((idx_rows, 128), jnp.int32),
            pltpu.VMEM((ch, CIN), jnp.float32),
            pltpu.SemaphoreType.DMA,
        ],
    )
    def gather_k(table_hbm, idx_hbm, out_hbm, idx_v, rows_v, sem):
        wid = lax.axis_index("s") * info.num_cores + lax.axis_index("c")
        pltpu.sync_copy(idx_hbm.at[pl.ds(wid * idx_rows, idx_rows)], idx_v)

        def body(c, carry):
            cps = [
                pltpu.make_async_copy(
                    table_hbm.at[idx_v.at[c * dmas + j]],
                    rows_v.at[pl.ds(j * 128, 128)],
                    sem,
                )
                for j in range(dmas)
            ]
            for cp in cps:
                cp.start()
            for cp in cps:
                cp.wait()
            base = wid * rows_w + c * ch
            pltpu.sync_copy(rows_v, out_hbm.at[pl.ds(base, ch)])
            return carry

        lax.fori_loop(0, nch, body, 0)

    return gather_k(table, idx2d)


STATS_ROWS = 8192
MAIN_NP = 256
FIN_NP = 4096


def _stats_body(x_ref, w_ref, b_ref, acc_ref):
    i = pl.program_id(0)
    y = jnp.dot(x_ref[...], w_ref[...], preferred_element_type=jnp.float32)
    y = y + b_ref[...]
    upd = jnp.concatenate(
        [
            jnp.sum(y, axis=0, keepdims=True),
            jnp.sum(y * y, axis=0, keepdims=True),
            jnp.zeros((6, CREL), jnp.float32),
        ],
        axis=0,
    )

    @pl.when(i == 0)
    def _():
        acc_ref[...] = upd

    @pl.when(i > 0)
    def _():
        acc_ref[...] += upd


def _main_body(fg_ref, orel_ref, feat_ref, w1_ref, b1_ref, wat_ref,
               wout_ref, bout_ref, wsc_ref, bsc_ref,
               zout_ref, zsc_ref, acc_ref):
    i = pl.program_id(0)
    nrk = MAIN_NP * K
    y = jnp.dot(orel_ref[...], w1_ref[...], preferred_element_type=jnp.float32)
    y = y + b1_ref[...]
    rel = jnp.where(y >= 0, y, 0.2 * y)
    f = jnp.concatenate([fg_ref[...], rel], axis=1)
    logits = jnp.dot(f, wat_ref[...], preferred_element_type=jnp.float32)
    l3 = logits.reshape(MAIN_NP, K, CIN + CREL)
    m = jnp.max(l3, axis=1, keepdims=True)
    e = jnp.exp(l3 - m)
    attn = e / jnp.sum(e, axis=1, keepdims=True)
    f3 = f.reshape(MAIN_NP, K, CIN + CREL)
    pooled = jnp.sum(attn * f3, axis=1)
    z_out = jnp.dot(pooled, wout_ref[...], preferred_element_type=jnp.float32)
    z_out = z_out + bout_ref[...]
    z_sc = jnp.dot(feat_ref[...], wsc_ref[...], preferred_element_type=jnp.float32)
    z_sc = z_sc + bsc_ref[...]
    zout_ref[...] = z_out
    zsc_ref[...] = z_sc
    upd = jnp.concatenate(
        [
            jnp.sum(z_out, axis=0, keepdims=True),
            jnp.sum(z_out * z_out, axis=0, keepdims=True),
            jnp.sum(z_sc, axis=0, keepdims=True),
            jnp.sum(z_sc * z_sc, axis=0, keepdims=True),
            jnp.zeros((4, COUT), jnp.float32),
        ],
        axis=0,
    )

    @pl.when(i == 0)
    def _():
        acc_ref[...] = upd

    @pl.when(i > 0)
    def _():
        acc_ref[...] += upd


def _final_body(zout_ref, zsc_ref, coef_ref, out_ref):
    a_out = coef_ref[0:1, :]
    c_out = coef_ref[1:2, :]
    a_sc = coef_ref[2:3, :]
    c_sc = coef_ref[3:4, :]
    y = zsc_ref[...] * a_sc + c_sc + zout_ref[...] * a_out + c_out
    out_ref[...] = jnp.where(y >= 0, y, 0.2 * y)


def kernel(xyz, feature, ori_relative_feature, neighbors_idx,
           W1, b1, g1, be1, W_attn, W_out, b_out, g_out, be_out,
           W_sc, b_sc, g_sc, be_sc):
    feat2 = feature.reshape(P, CIN)
    orel2 = ori_relative_feature.reshape(R, 10)
    offs = (jnp.arange(B, dtype=jnp.int32) * N)[:, None, None]
    idx2d = (neighbors_idx + offs).reshape(R // 128, 128)

    fg = _sc_gather(feat2, idx2d)

    sums1 = pl.pallas_call(
        _stats_body,
        grid=(R // STATS_ROWS,),
        in_specs=[
            pl.BlockSpec((STATS_ROWS, 10), lambda i: (i, 0)),
            pl.BlockSpec((10, CREL), lambda i: (0, 0)),
            pl.BlockSpec((1, CREL), lambda i: (0, 0)),
        ],
        out_specs=pl.BlockSpec((8, CREL), lambda i: (0, 0)),
        out_shape=jax.ShapeDtypeStruct((8, CREL), jnp.float32),
    )(orel2, W1, b1.reshape(1, CREL))

    m1 = sums1[0] / R
    v1 = sums1[1] / R - m1 * m1
    a1 = g1 * lax.rsqrt(v1 + EPS)
    w1f = W1 * a1[None, :]
    b1f = ((b1 - m1) * a1 + be1).reshape(1, CREL)

    nrk = MAIN_NP * K
    z_out, z_sc, sums2 = pl.pallas_call(
        _main_body,
        grid=(P // MAIN_NP,),
        in_specs=[
            pl.BlockSpec((nrk, CIN), lambda i: (i, 0)),
            pl.BlockSpec((nrk, 10), lambda i: (i, 0)),
            pl.BlockSpec((MAIN_NP, CIN), lambda i: (i, 0)),
            pl.BlockSpec((10, CREL), lambda i: (0, 0)),
            pl.BlockSpec((1, CREL), lambda i: (0, 0)),
            pl.BlockSpec((CIN + CREL, CIN + CREL), lambda i: (0, 0)),
            pl.BlockSpec((CIN + CREL, COUT), lambda i: (0, 0)),
            pl.BlockSpec((1, COUT), lambda i: (0, 0)),
            pl.BlockSpec((CIN, COUT), lambda i: (0, 0)),
            pl.BlockSpec((1, COUT), lambda i: (0, 0)),
        ],
        out_specs=[
            pl.BlockSpec((MAIN_NP, COUT), lambda i: (i, 0)),
            pl.BlockSpec((MAIN_NP, COUT), lambda i: (i, 0)),
            pl.BlockSpec((8, COUT), lambda i: (0, 0)),
        ],
        out_shape=[
            jax.ShapeDtypeStruct((P, COUT), jnp.float32),
            jax.ShapeDtypeStruct((P, COUT), jnp.float32),
            jax.ShapeDtypeStruct((8, COUT), jnp.float32),
        ],
    )(fg, orel2, feat2, w1f, b1f, W_attn, W_out, b_out.reshape(1, COUT),
      W_sc, b_sc.reshape(1, COUT))

    mo = sums2[0] / P
    vo = sums2[1] / P - mo * mo
    ao = g_out * lax.rsqrt(vo + EPS)
    co = be_out - mo * ao
    ms = sums2[2] / P
    vs = sums2[3] / P - ms * ms
    asc = g_sc * lax.rsqrt(vs + EPS)
    csc = be_sc - ms * asc
    coef = jnp.concatenate(
        [jnp.stack([ao, co, asc, csc]), jnp.zeros((4, COUT), jnp.float32)], axis=0)

    out = pl.pallas_call(
        _final_body,
        grid=(P // FIN_NP,),
        in_specs=[
            pl.BlockSpec((FIN_NP, COUT), lambda i: (i, 0)),
            pl.BlockSpec((FIN_NP, COUT), lambda i: (i, 0)),
            pl.BlockSpec((8, COUT), lambda i: (0, 0)),
        ],
        out_specs=pl.BlockSpec((FIN_NP, COUT), lambda i: (i, 0)),
        out_shape=jax.ShapeDtypeStruct((P, COUT), jnp.float32),
    )(z_out, z_sc, coef)

    return (xyz, out.reshape(B, N, COUT), ori_relative_feature, neighbors_idx)


# speedup vs baseline: 18.3125x; 1.3538x over previous
import functools

import jax
import jax.numpy as jnp
from jax import lax
from jax.experimental import pallas as pl
from jax.experimental.pallas import tpu as pltpu
from jax.experimental.pallas import tpu_sc as plsc

B, N, K = 4, 16384, 16
CIN, CREL, COUT = 32, 32, 64
P = B * N
R = P * K
RP = R // 4
EPS = 1e-5


def _sc_gather(table, idx2d):
    info = plsc.get_sparse_core_info()
    nw = info.num_cores * info.num_subcores
    rows_w = R // nw
    idx_rows = rows_w // 128
    ch = 2048
    nch = rows_w // ch
    dmas = ch // 128
    mesh = plsc.VectorSubcoreMesh(core_axis_name="c", subcore_axis_name="s")

    @functools.partial(
        pl.kernel,
        mesh=mesh,
        compiler_params=pltpu.CompilerParams(use_tc_tiling_on_sc=False),
        out_type=jax.ShapeDtypeStruct((R, CIN), jnp.float32),
        scratch_types=[
            pltpu.VMEM((idx_rows, 128), jnp.int32),
            pltpu.VMEM((ch, CIN), jnp.float32),
            pltpu.SemaphoreType.DMA,
        ],
    )
    def gather_k(table_hbm, idx_hbm, out_hbm, idx_v, rows_v, sem):
        wid = lax.axis_index("s") * info.num_cores + lax.axis_index("c")
        pltpu.sync_copy(idx_hbm.at[pl.ds(wid * idx_rows, idx_rows)], idx_v)

        def body(c, carry):
            cps = [
                pltpu.make_async_copy(
                    table_hbm.at[idx_v.at[c * dmas + j]],
                    rows_v.at[pl.ds(j * 128, 128)],
                    sem,
                )
                for j in range(dmas)
            ]
            for cp in cps:
                cp.start()
            for cp in cps:
                cp.wait()
            base = wid * rows_w + c * ch
            pltpu.sync_copy(rows_v, out_hbm.at[pl.ds(base, ch)])
            return carry

        lax.fori_loop(0, nch, body, 0)

    return gather_k(table, idx2d)


A_ROWS = 16384
MAIN_NP = 1024
FIN_NP = 4096


def _passa_body(x_ref, w_ref, b_ref, yp_ref, acc_ref):
    i = pl.program_id(0)
    q = A_ROWS // 4
    y = jnp.dot(x_ref[...], w_ref[...], preferred_element_type=jnp.float32)
    y = y + b_ref[...]
    yp_ref[...] = jnp.concatenate(
        [y[0:q], y[q:2 * q], y[2 * q:3 * q], y[3 * q:4 * q]], axis=1)
    upd = jnp.concatenate(
        [
            jnp.sum(y, axis=0, keepdims=True),
            jnp.sum(y * y, axis=0, keepdims=True),
            jnp.zeros((6, CREL), jnp.float32),
        ],
        axis=0,
    )

    @pl.when(i == 0)
    def _():
        acc_ref[...] = upd

    @pl.when(i > 0)
    def _():
        acc_ref[...] += upd


def _main_body(fgp_ref, yp_ref, feat_ref, a1_ref, c1_ref, wat_ref,
               wout_ref, bout_ref, wsc_ref, bsc_ref,
               zout_ref, zsc_ref, acc_ref):
    i = pl.program_id(0)
    npk = MAIN_NP * K // 4
    npj = MAIN_NP // 4
    fgp = fgp_ref[...]
    relp = yp_ref[...] * a1_ref[...] + c1_ref[...]
    relp = jnp.where(relp >= 0, relp, 0.2 * relp)
    pooled_parts = []
    for j in range(4):
        fj = jnp.concatenate(
            [fgp[:, 32 * j:32 * j + 32], relp[:, 32 * j:32 * j + 32]], axis=1)
        logits = jnp.dot(fj, wat_ref[...], preferred_element_type=jnp.float32)
        l3 = logits.reshape(npj, K, CIN + CREL)
        m = jnp.max(l3, axis=1, keepdims=True)
        e = jnp.exp(l3 - m)
        attn = e / jnp.sum(e, axis=1, keepdims=True)
        f3 = fj.reshape(npj, K, CIN + CREL)
        pooled_parts.append(jnp.sum(attn * f3, axis=1))
    pooled = jnp.concatenate(pooled_parts, axis=0)
    z_out = jnp.dot(pooled, wout_ref[...], preferred_element_type=jnp.float32)
    z_out = z_out + bout_ref[...]
    z_sc = jnp.dot(feat_ref[...], wsc_ref[...], preferred_element_type=jnp.float32)
    z_sc = z_sc + bsc_ref[...]
    zout_ref[...] = z_out
    zsc_ref[...] = z_sc
    upd = jnp.concatenate(
        [
            jnp.sum(z_out, axis=0, keepdims=True),
            jnp.sum(z_out * z_out, axis=0, keepdims=True),
            jnp.sum(z_sc, axis=0, keepdims=True),
            jnp.sum(z_sc * z_sc, axis=0, keepdims=True),
            jnp.zeros((4, COUT), jnp.float32),
        ],
        axis=0,
    )

    @pl.when(i == 0)
    def _():
        acc_ref[...] = upd

    @pl.when(i > 0)
    def _():
        acc_ref[...] += upd


def _final_body(zout_ref, zsc_ref, coef_ref, out_ref):
    a_out = coef_ref[0:1, :]
    c_out = coef_ref[1:2, :]
    a_sc = coef_ref[2:3, :]
    c_sc = coef_ref[3:4, :]
    y = zsc_ref[...] * a_sc + c_sc + zout_ref[...] * a_out + c_out
    out_ref[...] = jnp.where(y >= 0, y, 0.2 * y)


def kernel(xyz, feature, ori_relative_feature, neighbors_idx,
           W1, b1, g1, be1, W_attn, W_out, b_out, g_out, be_out,
           W_sc, b_sc, g_sc, be_sc):
    feat2 = feature.reshape(P, CIN)
    orel2 = ori_relative_feature.reshape(R, 10)
    offs = (jnp.arange(B, dtype=jnp.int32) * N)[:, None, None]
    nrk = MAIN_NP * K
    npk = nrk // 4
    idx2d = ((neighbors_idx + offs).reshape(R // nrk, 4, npk)
             .transpose(0, 2, 1).reshape(R // 128, 128))

    fgp = _sc_gather(feat2, idx2d).reshape(RP, 128)

    yp, sums1 = pl.pallas_call(
        _passa_body,
        grid=(R // A_ROWS,),
        in_specs=[
            pl.BlockSpec((A_ROWS, 10), lambda i: (i, 0)),
            pl.BlockSpec((10, CREL), lambda i: (0, 0)),
            pl.BlockSpec((1, CREL), lambda i: (0, 0)),
        ],
        out_specs=[
            pl.BlockSpec((A_ROWS // 4, 128), lambda i: (i, 0)),
            pl.BlockSpec((8, CREL), lambda i: (0, 0)),
        ],
        out_shape=[
            jax.ShapeDtypeStruct((RP, 128), jnp.float32),
            jax.ShapeDtypeStruct((8, CREL), jnp.float32),
        ],
    )(orel2, W1, b1.reshape(1, CREL))

    m1 = sums1[0] / R
    v1 = sums1[1] / R - m1 * m1
    a1 = g1 * lax.rsqrt(v1 + EPS)
    c1 = be1 - m1 * a1
    a1t = jnp.tile(a1, 4).reshape(1, 128)
    c1t = jnp.tile(c1, 4).reshape(1, 128)

    npk = MAIN_NP * K // 4
    z_out, z_sc, sums2 = pl.pallas_call(
        _main_body,
        grid=(P // MAIN_NP,),
        in_specs=[
            pl.BlockSpec((npk, 128), lambda i: (i, 0)),
            pl.BlockSpec((npk, 128), lambda i: (i, 0)),
            pl.BlockSpec((MAIN_NP, CIN), lambda i: (i, 0)),
            pl.BlockSpec((1, 128), lambda i: (0, 0)),
            pl.BlockSpec((1, 128), lambda i: (0, 0)),
            pl.BlockSpec((CIN + CREL, CIN + CREL), lambda i: (0, 0)),
            pl.BlockSpec((CIN + CREL, COUT), lambda i: (0, 0)),
            pl.BlockSpec((1, COUT), lambda i: (0, 0)),
            pl.BlockSpec((CIN, COUT), lambda i: (0, 0)),
            pl.BlockSpec((1, COUT), lambda i: (0, 0)),
        ],
        out_specs=[
            pl.BlockSpec((MAIN_NP, COUT), lambda i: (i, 0)),
            pl.BlockSpec((MAIN_NP, COUT), lambda i: (i, 0)),
            pl.BlockSpec((8, COUT), lambda i: (0, 0)),
        ],
        out_shape=[
            jax.ShapeDtypeStruct((P, COUT), jnp.float32),
            jax.ShapeDtypeStruct((P, COUT), jnp.float32),
            jax.ShapeDtypeStruct((8, COUT), jnp.float32),
        ],
    )(fgp, yp, feat2, a1t, c1t, W_attn, W_out, b_out.reshape(1, COUT),
      W_sc, b_sc.reshape(1, COUT))

    mo = sums2[0] / P
    vo = sums2[1] / P - mo * mo
    ao = g_out * lax.rsqrt(vo + EPS)
    co = be_out - mo * ao
    ms = sums2[2] / P
    vs = sums2[3] / P - ms * ms
    asc = g_sc * lax.rsqrt(vs + EPS)
    csc = be_sc - ms * asc
    coef = jnp.concatenate(
        [jnp.stack([ao, co, asc, csc]), jnp.zeros((4, COUT), jnp.float32)], axis=0)

    out = pl.pallas_call(
        _final_body,
        grid=(P // FIN_NP,),
        in_specs=[
            pl.BlockSpec((FIN_NP, COUT), lambda i: (i, 0)),
            pl.BlockSpec((FIN_NP, COUT), lambda i: (i, 0)),
            pl.BlockSpec((8, COUT), lambda i: (0, 0)),
        ],
        out_specs=pl.BlockSpec((FIN_NP, COUT), lambda i: (i, 0)),
        out_shape=jax.ShapeDtypeStruct((P, COUT), jnp.float32),
    )(z_out, z_sc, coef)

    return (xyz, out.reshape(B, N, COUT), ori_relative_feature, neighbors_idx)


# speedup vs baseline: 18.7864x; 1.0259x over previous
import functools

import jax
import jax.numpy as jnp
from jax import lax
from jax.experimental import pallas as pl
from jax.experimental.pallas import tpu as pltpu
from jax.experimental.pallas import tpu_sc as plsc

B, N, K = 4, 16384, 16
CIN, CREL, COUT = 32, 32, 64
P = B * N
R = P * K
RP = R // 4
EPS = 1e-5


def _sc_gather(table, idx2d):
    info = plsc.get_sparse_core_info()
    nw = info.num_cores * info.num_subcores
    rows_w = R // nw
    idx_rows = rows_w // 128
    ch = 2048
    nch = rows_w // ch
    dmas = ch // 128
    mesh = plsc.VectorSubcoreMesh(core_axis_name="c", subcore_axis_name="s")

    @functools.partial(
        pl.kernel,
        mesh=mesh,
        compiler_params=pltpu.CompilerParams(use_tc_tiling_on_sc=False),
        out_type=jax.ShapeDtypeStruct((R, CIN), jnp.float32),
        scratch_types=[
            pltpu.VMEM((idx_rows, 128), jnp.int32),
            pltpu.VMEM((ch, CIN), jnp.float32),
            pltpu.SemaphoreType.DMA,
        ],
    )
    def gather_k(table_hbm, idx_hbm, out_hbm, idx_v, rows_v, sem):
        wid = lax.axis_index("s") * info.num_cores + lax.axis_index("c")
        pltpu.sync_copy(idx_hbm.at[pl.ds(wid * idx_rows, idx_rows)], idx_v)

        def body(c, carry):
            cps = [
                pltpu.make_async_copy(
                    table_hbm.at[idx_v.at[c * dmas + j]],
                    rows_v.at[pl.ds(j * 128, 128)],
                    sem,
                )
                for j in range(dmas)
            ]
            for cp in cps:
                cp.start()
            for cp in cps:
                cp.wait()
            base = wid * rows_w + c * ch
            pltpu.sync_copy(rows_v, out_hbm.at[pl.ds(base, ch)])
            return carry

        lax.fori_loop(0, nch, body, 0)

    return gather_k(table, idx2d)


A_COLS = 4096
MAIN_NP = 1024
FIN_NP = 4096
NT = MAIN_NP // 4


def _passa_body(x_ref, w_ref, b_ref, yp_ref, acc_ref):
    first = ((pl.program_id(0) == 0) & (pl.program_id(1) == 0)
             & (pl.program_id(2) == 0))
    x = x_ref[...].reshape(10, A_COLS)
    y = lax.dot_general(x, w_ref[...], (((0,), (0,)), ((), ())),
                        preferred_element_type=jnp.float32)
    y = y + b_ref[...]
    parts = []
    for s in range(A_COLS // MAIN_NP):
        ys = y[s * MAIN_NP:(s + 1) * MAIN_NP]
        parts.append(jnp.concatenate(
            [ys[0:NT], ys[NT:2 * NT], ys[2 * NT:3 * NT], ys[3 * NT:4 * NT]],
            axis=1)[None])
    yp_ref[...] = jnp.concatenate(parts, axis=0)[None]
    upd = jnp.concatenate(
        [
            jnp.sum(y, axis=0, keepdims=True),
            jnp.sum(y * y, axis=0, keepdims=True),
            jnp.zeros((6, CREL), jnp.float32),
        ],
        axis=0,
    )

    @pl.when(first)
    def _():
        acc_ref[...] = upd

    @pl.when(jnp.logical_not(first))
    def _():
        acc_ref[...] += upd


def _main_body(fgp_ref, yp_ref, feat_ref, a1_ref, c1_ref, wat_ref,
               wout_ref, bout_ref, wsc_ref, bsc_ref,
               zout_ref, zsc_ref, acc_ref):
    i = pl.program_id(0)
    npk = MAIN_NP * K // 4
    fgp = fgp_ref[...]
    relp = yp_ref[...].reshape(npk, 128) * a1_ref[...] + c1_ref[...]
    relp = jnp.where(relp >= 0, relp, 0.2 * relp)
    pooled_parts = []
    for j in range(4):
        fj = jnp.concatenate(
            [fgp[:, 32 * j:32 * j + 32], relp[:, 32 * j:32 * j + 32]], axis=1)
        logits = jnp.dot(fj, wat_ref[...], preferred_element_type=jnp.float32)
        l3 = logits.reshape(K, NT, CIN + CREL)
        m = jnp.max(l3, axis=0, keepdims=True)
        e = jnp.exp(l3 - m)
        attn = e / jnp.sum(e, axis=0, keepdims=True)
        f3 = fj.reshape(K, NT, CIN + CREL)
        pooled_parts.append(jnp.sum(attn * f3, axis=0))
    pooled = jnp.concatenate(pooled_parts, axis=0)
    z_out = jnp.dot(pooled, wout_ref[...], preferred_element_type=jnp.float32)
    z_out = z_out + bout_ref[...]
    z_sc = jnp.dot(feat_ref[...], wsc_ref[...], preferred_element_type=jnp.float32)
    z_sc = z_sc + bsc_ref[...]
    zout_ref[...] = z_out
    zsc_ref[...] = z_sc
    upd = jnp.concatenate(
        [
            jnp.sum(z_out, axis=0, keepdims=True),
            jnp.sum(z_out * z_out, axis=0, keepdims=True),
            jnp.sum(z_sc, axis=0, keepdims=True),
            jnp.sum(z_sc * z_sc, axis=0, keepdims=True),
            jnp.zeros((4, COUT), jnp.float32),
        ],
        axis=0,
    )

    @pl.when(i == 0)
    def _():
        acc_ref[...] = upd

    @pl.when(i > 0)
    def _():
        acc_ref[...] += upd


def _final_body(zout_ref, zsc_ref, coef_ref, out_ref):
    a_out = coef_ref[0:1, :]
    c_out = coef_ref[1:2, :]
    a_sc = coef_ref[2:3, :]
    c_sc = coef_ref[3:4, :]
    y = zsc_ref[...] * a_sc + c_sc + zout_ref[...] * a_out + c_out
    out_ref[...] = jnp.where(y >= 0, y, 0.2 * y)


def kernel(xyz, feature, ori_relative_feature, neighbors_idx,
           W1, b1, g1, be1, W_attn, W_out, b_out, g_out, be_out,
           W_sc, b_sc, g_sc, be_sc):
    feat2 = feature.reshape(P, CIN)
    orelT = jnp.transpose(ori_relative_feature, (0, 3, 2, 1)).reshape(B, 10, K * N)
    offs = (jnp.arange(B, dtype=jnp.int32) * N)[:, None, None]
    nrk = MAIN_NP * K
    nb = R // nrk
    idx2d = ((neighbors_idx + offs).reshape(nb, 4, NT, K)
             .transpose(0, 3, 2, 1).reshape(R // 128, 128))

    fgp = _sc_gather(feat2, idx2d).reshape(RP, 128)

    nseg = N // MAIN_NP
    ncs = nseg // (A_COLS // MAIN_NP)
    yp4, sums1 = pl.pallas_call(
        _passa_body,
        grid=(B, K, ncs),
        in_specs=[
            pl.BlockSpec((1, 10, A_COLS), lambda b, k, c: (b, 0, k * ncs + c)),
            pl.BlockSpec((10, CREL), lambda b, k, c: (0, 0)),
            pl.BlockSpec((1, CREL), lambda b, k, c: (0, 0)),
        ],
        out_specs=[
            pl.BlockSpec((1, A_COLS // MAIN_NP, NT, 128),
                         lambda b, k, c: (k, b * ncs + c, 0, 0)),
            pl.BlockSpec((8, CREL), lambda b, k, c: (0, 0)),
        ],
        out_shape=[
            jax.ShapeDtypeStruct((K, nb, NT, 128), jnp.float32),
            jax.ShapeDtypeStruct((8, CREL), jnp.float32),
        ],
    )(orelT, W1, b1.reshape(1, CREL))

    m1 = sums1[0] / R
    v1 = sums1[1] / R - m1 * m1
    a1 = g1 * lax.rsqrt(v1 + EPS)
    c1 = be1 - m1 * a1
    a1t = jnp.tile(a1, 4).reshape(1, 128)
    c1t = jnp.tile(c1, 4).reshape(1, 128)

    npk = MAIN_NP * K // 4
    z_out, z_sc, sums2 = pl.pallas_call(
        _main_body,
        grid=(P // MAIN_NP,),
        in_specs=[
            pl.BlockSpec((npk, 128), lambda i: (i, 0)),
            pl.BlockSpec((K, 1, NT, 128), lambda i: (0, i, 0, 0)),
            pl.BlockSpec((MAIN_NP, CIN), lambda i: (i, 0)),
            pl.BlockSpec((1, 128), lambda i: (0, 0)),
            pl.BlockSpec((1, 128), lambda i: (0, 0)),
            pl.BlockSpec((CIN + CREL, CIN + CREL), lambda i: (0, 0)),
            pl.BlockSpec((CIN + CREL, COUT), lambda i: (0, 0)),
            pl.BlockSpec((1, COUT), lambda i: (0, 0)),
            pl.BlockSpec((CIN, COUT), lambda i: (0, 0)),
            pl.BlockSpec((1, COUT), lambda i: (0, 0)),
        ],
        out_specs=[
            pl.BlockSpec((MAIN_NP, COUT), lambda i: (i, 0)),
            pl.BlockSpec((MAIN_NP, COUT), lambda i: (i, 0)),
            pl.BlockSpec((8, COUT), lambda i: (0, 0)),
        ],
        out_shape=[
            jax.ShapeDtypeStruct((P, COUT), jnp.float32),
            jax.ShapeDtypeStruct((P, COUT), jnp.float32),
            jax.ShapeDtypeStruct((8, COUT), jnp.float32),
        ],
    )(fgp, yp4, feat2, a1t, c1t, W_attn, W_out, b_out.reshape(1, COUT),
      W_sc, b_sc.reshape(1, COUT))

    mo = sums2[0] / P
    vo = sums2[1] / P - mo * mo
    ao = g_out * lax.rsqrt(vo + EPS)
    co = be_out - mo * ao
    ms = sums2[2] / P
    vs = sums2[3] / P - ms * ms
    asc = g_sc * lax.rsqrt(vs + EPS)
    csc = be_sc - ms * asc
    coef = jnp.concatenate(
        [jnp.stack([ao, co, asc, csc]), jnp.zeros((4, COUT), jnp.float32)], axis=0)

    out = pl.pallas_call(
        _final_body,
        grid=(P // FIN_NP,),
        in_specs=[
            pl.BlockSpec((FIN_NP, COUT), lambda i: (i, 0)),
            pl.BlockSpec((FIN_NP, COUT), lambda i: (i, 0)),
            pl.BlockSpec((8, COUT), lambda i: (0, 0)),
        ],
        out_specs=pl.BlockSpec((FIN_NP, COUT), lambda i: (i, 0)),
        out_shape=jax.ShapeDtypeStruct((P, COUT), jnp.float32),
    )(z_out, z_sc, coef)

    return (xyz, out.reshape(B, N, COUT), ori_relative_feature, neighbors_idx)


# speedup vs baseline: 19.5755x; 1.0420x over previous
import functools

import jax
import jax.numpy as jnp
from jax import lax
from jax.experimental import pallas as pl
from jax.experimental.pallas import tpu as pltpu
from jax.experimental.pallas import tpu_sc as plsc

B, N, K = 4, 16384, 16
CIN, CREL, COUT = 32, 32, 64
P = B * N
R = P * K
RP = R // 4
EPS = 1e-5


def _sc_gather(table, idx2d):
    info = plsc.get_sparse_core_info()
    nw = info.num_cores * info.num_subcores
    rows_w = R // nw
    idx_rows = rows_w // 128
    ch = 2048
    nch = rows_w // ch
    dmas = ch // 128
    mesh = plsc.VectorSubcoreMesh(core_axis_name="c", subcore_axis_name="s")

    @functools.partial(
        pl.kernel,
        mesh=mesh,
        compiler_params=pltpu.CompilerParams(use_tc_tiling_on_sc=False),
        out_type=jax.ShapeDtypeStruct((R, CIN), jnp.float32),
        scratch_types=[
            pltpu.VMEM((idx_rows, 128), jnp.int32),
            pltpu.VMEM((ch, CIN), jnp.float32),
            pltpu.SemaphoreType.DMA,
        ],
    )
    def gather_k(table_hbm, idx_hbm, out_hbm, idx_v, rows_v, sem):
        wid = lax.axis_index("s") * info.num_cores + lax.axis_index("c")
        pltpu.sync_copy(idx_hbm.at[pl.ds(wid * idx_rows, idx_rows)], idx_v)

        def body(c, carry):
            cps = [
                pltpu.make_async_copy(
                    table_hbm.at[idx_v.at[c * dmas + j]],
                    rows_v.at[pl.ds(j * 128, 128)],
                    sem,
                )
                for j in range(dmas)
            ]
            for cp in cps:
                cp.start()
            for cp in cps:
                cp.wait()
            base = wid * rows_w + c * ch
            pltpu.sync_copy(rows_v, out_hbm.at[pl.ds(base, ch)])
            return carry

        lax.fori_loop(0, nch, body, 0)

    return gather_k(table, idx2d)


A_COLS = 16384
MAIN_NP = 1024
FIN_NP = 4096
NT = MAIN_NP // 4


def _passa_body(x_ref, w_ref, b_ref, yp_ref, acc_ref):
    first = ((pl.program_id(0) == 0) & (pl.program_id(1) == 0)
             & (pl.program_id(2) == 0))
    x = x_ref[...].reshape(10, A_COLS)
    y = lax.dot_general(x, w_ref[...], (((0,), (0,)), ((), ())),
                        preferred_element_type=jnp.float32)
    y = y + b_ref[...]
    parts = []
    for s in range(A_COLS // MAIN_NP):
        ys = y[s * MAIN_NP:(s + 1) * MAIN_NP]
        parts.append(jnp.concatenate(
            [ys[0:NT], ys[NT:2 * NT], ys[2 * NT:3 * NT], ys[3 * NT:4 * NT]],
            axis=1)[None])
    yp_ref[...] = jnp.concatenate(parts, axis=0)[None]
    upd = jnp.concatenate(
        [
            jnp.sum(y, axis=0, keepdims=True),
            jnp.sum(y * y, axis=0, keepdims=True),
            jnp.zeros((6, CREL), jnp.float32),
        ],
        axis=0,
    )

    @pl.when(first)
    def _():
        acc_ref[...] = upd

    @pl.when(jnp.logical_not(first))
    def _():
        acc_ref[...] += upd


def _main_body(fgp_ref, yp_ref, feat_ref, a1_ref, c1_ref, wat_ref,
               wout_ref, bout_ref, wsc_ref, bsc_ref,
               zout_ref, zsc_ref, acc_ref):
    i = pl.program_id(0)
    npk = MAIN_NP * K // 4
    fgp = fgp_ref[...]
    relp = yp_ref[...].reshape(npk, 128) * a1_ref[...] + c1_ref[...]
    relp = jnp.where(relp >= 0, relp, 0.2 * relp)
    pooled_parts = []
    for j in range(4):
        fj = jnp.concatenate(
            [fgp[:, 32 * j:32 * j + 32], relp[:, 32 * j:32 * j + 32]], axis=1)
        logits = jnp.dot(fj, wat_ref[...], preferred_element_type=jnp.float32)
        l3 = logits.reshape(K, NT, CIN + CREL)
        m = jnp.max(l3, axis=0, keepdims=True)
        e = jnp.exp(l3 - m)
        f3 = fj.reshape(K, NT, CIN + CREL)
        raw = jnp.sum(e * f3, axis=0)
        pooled_parts.append(raw / jnp.sum(e, axis=0))
    pooled = jnp.concatenate(pooled_parts, axis=0)
    z_out = jnp.dot(pooled, wout_ref[...], preferred_element_type=jnp.float32)
    z_out = z_out + bout_ref[...]
    z_sc = jnp.dot(feat_ref[...], wsc_ref[...], preferred_element_type=jnp.float32)
    z_sc = z_sc + bsc_ref[...]
    zout_ref[...] = z_out
    zsc_ref[...] = z_sc
    upd = jnp.concatenate(
        [
            jnp.sum(z_out, axis=0, keepdims=True),
            jnp.sum(z_out * z_out, axis=0, keepdims=True),
            jnp.sum(z_sc, axis=0, keepdims=True),
            jnp.sum(z_sc * z_sc, axis=0, keepdims=True),
            jnp.zeros((4, COUT), jnp.float32),
        ],
        axis=0,
    )

    @pl.when(i == 0)
    def _():
        acc_ref[...] = upd

    @pl.when(i > 0)
    def _():
        acc_ref[...] += upd


def _final_body(zout_ref, zsc_ref, coef_ref, out_ref):
    a_out = coef_ref[0:1, :]
    c_out = coef_ref[1:2, :]
    a_sc = coef_ref[2:3, :]
    c_sc = coef_ref[3:4, :]
    y = zsc_ref[...] * a_sc + c_sc + zout_ref[...] * a_out + c_out
    out_ref[...] = jnp.where(y >= 0, y, 0.2 * y)


def kernel(xyz, feature, ori_relative_feature, neighbors_idx,
           W1, b1, g1, be1, W_attn, W_out, b_out, g_out, be_out,
           W_sc, b_sc, g_sc, be_sc):
    feat2 = feature.reshape(P, CIN)
    orelT = jnp.transpose(ori_relative_feature, (0, 3, 2, 1)).reshape(B, 10, K * N)
    offs = (jnp.arange(B, dtype=jnp.int32) * N)[:, None, None]
    nrk = MAIN_NP * K
    nb = R // nrk
    idx2d = ((neighbors_idx + offs).reshape(nb, 4, NT, K)
             .transpose(0, 3, 2, 1).reshape(R // 128, 128))

    fgp = _sc_gather(feat2, idx2d).reshape(RP, 128)

    nseg = N // MAIN_NP
    ncs = nseg // (A_COLS // MAIN_NP)
    yp4, sums1 = pl.pallas_call(
        _passa_body,
        grid=(B, K, ncs),
        in_specs=[
            pl.BlockSpec((1, 10, A_COLS), lambda b, k, c: (b, 0, k * ncs + c)),
            pl.BlockSpec((10, CREL), lambda b, k, c: (0, 0)),
            pl.BlockSpec((1, CREL), lambda b, k, c: (0, 0)),
        ],
        out_specs=[
            pl.BlockSpec((1, A_COLS // MAIN_NP, NT, 128),
                         lambda b, k, c: (k, b * ncs + c, 0, 0)),
            pl.BlockSpec((8, CREL), lambda b, k, c: (0, 0)),
        ],
        out_shape=[
            jax.ShapeDtypeStruct((K, nb, NT, 128), jnp.float32),
            jax.ShapeDtypeStruct((8, CREL), jnp.float32),
        ],
    )(orelT, W1, b1.reshape(1, CREL))

    m1 = sums1[0] / R
    v1 = sums1[1] / R - m1 * m1
    a1 = g1 * lax.rsqrt(v1 + EPS)
    c1 = be1 - m1 * a1
    a1t = jnp.tile(a1, 4).reshape(1, 128)
    c1t = jnp.tile(c1, 4).reshape(1, 128)

    npk = MAIN_NP * K // 4
    z_out, z_sc, sums2 = pl.pallas_call(
        _main_body,
        grid=(P // MAIN_NP,),
        in_specs=[
            pl.BlockSpec((npk, 128), lambda i: (i, 0)),
            pl.BlockSpec((K, 1, NT, 128), lambda i: (0, i, 0, 0)),
            pl.BlockSpec((MAIN_NP, CIN), lambda i: (i, 0)),
            pl.BlockSpec((1, 128), lambda i: (0, 0)),
            pl.BlockSpec((1, 128), lambda i: (0, 0)),
            pl.BlockSpec((CIN + CREL, CIN + CREL), lambda i: (0, 0)),
            pl.BlockSpec((CIN + CREL, COUT), lambda i: (0, 0)),
            pl.BlockSpec((1, COUT), lambda i: (0, 0)),
            pl.BlockSpec((CIN, COUT), lambda i: (0, 0)),
            pl.BlockSpec((1, COUT), lambda i: (0, 0)),
        ],
        out_specs=[
            pl.BlockSpec((MAIN_NP, COUT), lambda i: (i, 0)),
            pl.BlockSpec((MAIN_NP, COUT), lambda i: (i, 0)),
            pl.BlockSpec((8, COUT), lambda i: (0, 0)),
        ],
        out_shape=[
            jax.ShapeDtypeStruct((P, COUT), jnp.float32),
            jax.ShapeDtypeStruct((P, COUT), jnp.float32),
            jax.ShapeDtypeStruct((8, COUT), jnp.float32),
        ],
    )(fgp, yp4, feat2, a1t, c1t, W_attn, W_out, b_out.reshape(1, COUT),
      W_sc, b_sc.reshape(1, COUT))

    mo = sums2[0] / P
    vo = sums2[1] / P - mo * mo
    ao = g_out * lax.rsqrt(vo + EPS)
    co = be_out - mo * ao
    ms = sums2[2] / P
    vs = sums2[3] / P - ms * ms
    asc = g_sc * lax.rsqrt(vs + EPS)
    csc = be_sc - ms * asc
    coef = jnp.concatenate(
        [jnp.stack([ao, co, asc, csc]), jnp.zeros((4, COUT), jnp.float32)], axis=0)

    out = pl.pallas_call(
        _final_body,
        grid=(P // FIN_NP,),
        in_specs=[
            pl.BlockSpec((FIN_NP, COUT), lambda i: (i, 0)),
            pl.BlockSpec((FIN_NP, COUT), lambda i: (i, 0)),
            pl.BlockSpec((8, COUT), lambda i: (0, 0)),
        ],
        out_specs=pl.BlockSpec((FIN_NP, COUT), lambda i: (i, 0)),
        out_shape=jax.ShapeDtypeStruct((P, COUT), jnp.float32),
    )(z_out, z_sc, coef)

    return (xyz, out.reshape(B, N, COUT), ori_relative_feature, neighbors_idx)


# speedup vs baseline: 19.6129x; 1.0019x over previous
import functools

import jax
import jax.numpy as jnp
from jax import lax
from jax.experimental import pallas as pl
from jax.experimental.pallas import tpu as pltpu
from jax.experimental.pallas import tpu_sc as plsc

B, N, K = 4, 16384, 16
CIN, CREL, COUT = 32, 32, 64
P = B * N
R = P * K
RP = R // 4
EPS = 1e-5


def _sc_gather(table, idx2d):
    info = plsc.get_sparse_core_info()
    nw = info.num_cores * info.num_subcores
    rows_w = R // nw
    idx_rows = rows_w // 128
    ch = 2048
    nch = rows_w // ch
    dmas = ch // 128
    mesh = plsc.VectorSubcoreMesh(core_axis_name="c", subcore_axis_name="s")

    @functools.partial(
        pl.kernel,
        mesh=mesh,
        compiler_params=pltpu.CompilerParams(use_tc_tiling_on_sc=False),
        out_type=jax.ShapeDtypeStruct((R, CIN), jnp.float32),
        scratch_types=[
            pltpu.VMEM((idx_rows, 128), jnp.int32),
            pltpu.VMEM((ch, CIN), jnp.float32),
            pltpu.SemaphoreType.DMA,
        ],
    )
    def gather_k(table_hbm, idx_hbm, out_hbm, idx_v, rows_v, sem):
        wid = lax.axis_index("s") * info.num_cores + lax.axis_index("c")
        pltpu.sync_copy(idx_hbm.at[pl.ds(wid * idx_rows, idx_rows)], idx_v)

        def body(c, carry):
            cps = [
                pltpu.make_async_copy(
                    table_hbm.at[idx_v.at[c * dmas + j]],
                    rows_v.at[pl.ds(j * 128, 128)],
                    sem,
                )
                for j in range(dmas)
            ]
            for cp in cps:
                cp.start()
            for cp in cps:
                cp.wait()
            base = wid * rows_w + c * ch
            pltpu.sync_copy(rows_v, out_hbm.at[pl.ds(base, ch)])
            return carry

        lax.fori_loop(0, nch, body, 0)

    return gather_k(table, idx2d)


A_COLS = 16384
MAIN_NP = 2048
FIN_NP = 4096
NT = MAIN_NP // 4


def _passa_body(x_ref, w_ref, b_ref, yp_ref, acc_ref):
    first = ((pl.program_id(0) == 0) & (pl.program_id(1) == 0)
             & (pl.program_id(2) == 0))
    x = x_ref[...].reshape(10, A_COLS)
    y = lax.dot_general(x, w_ref[...], (((0,), (0,)), ((), ())),
                        preferred_element_type=jnp.float32)
    y = y + b_ref[...]
    parts = []
    for s in range(A_COLS // MAIN_NP):
        ys = y[s * MAIN_NP:(s + 1) * MAIN_NP]
        parts.append(jnp.concatenate(
            [ys[0:NT], ys[NT:2 * NT], ys[2 * NT:3 * NT], ys[3 * NT:4 * NT]],
            axis=1)[None])
    yp_ref[...] = jnp.concatenate(parts, axis=0)[None]
    upd = jnp.concatenate(
        [
            jnp.sum(y, axis=0, keepdims=True),
            jnp.sum(y * y, axis=0, keepdims=True),
            jnp.zeros((6, CREL), jnp.float32),
        ],
        axis=0,
    )

    @pl.when(first)
    def _():
        acc_ref[...] = upd

    @pl.when(jnp.logical_not(first))
    def _():
        acc_ref[...] += upd


def _main_body(fgp_ref, yp_ref, feat_ref, a1_ref, c1_ref, wat_ref,
               wout_ref, bout_ref, wsc_ref, bsc_ref,
               zout_ref, zsc_ref, acc_ref):
    i = pl.program_id(0)
    npk = MAIN_NP * K // 4
    fgp = fgp_ref[...]
    relp = yp_ref[...].reshape(npk, 128) * a1_ref[...] + c1_ref[...]
    relp = jnp.where(relp >= 0, relp, 0.2 * relp)
    pooled_parts = []
    for j in range(4):
        fj = jnp.concatenate(
            [fgp[:, 32 * j:32 * j + 32], relp[:, 32 * j:32 * j + 32]], axis=1)
        logits = jnp.dot(fj, wat_ref[...], preferred_element_type=jnp.float32)
        l3 = logits.reshape(K, NT, CIN + CREL)
        m = jnp.max(l3, axis=0, keepdims=True)
        e = jnp.exp(l3 - m)
        f3 = fj.reshape(K, NT, CIN + CREL)
        raw = jnp.sum(e * f3, axis=0)
        pooled_parts.append(raw / jnp.sum(e, axis=0))
    pooled = jnp.concatenate(pooled_parts, axis=0)
    z_out = jnp.dot(pooled, wout_ref[...], preferred_element_type=jnp.float32)
    z_out = z_out + bout_ref[...]
    z_sc = jnp.dot(feat_ref[...], wsc_ref[...], preferred_element_type=jnp.float32)
    z_sc = z_sc + bsc_ref[...]
    zout_ref[...] = z_out
    zsc_ref[...] = z_sc
    upd = jnp.concatenate(
        [
            jnp.sum(z_out, axis=0, keepdims=True),
            jnp.sum(z_out * z_out, axis=0, keepdims=True),
            jnp.sum(z_sc, axis=0, keepdims=True),
            jnp.sum(z_sc * z_sc, axis=0, keepdims=True),
            jnp.zeros((4, COUT), jnp.float32),
        ],
        axis=0,
    )

    @pl.when(i == 0)
    def _():
        acc_ref[...] = upd

    @pl.when(i > 0)
    def _():
        acc_ref[...] += upd


def _final_body(zout_ref, zsc_ref, coef_ref, out_ref):
    a_out = coef_ref[0:1, :]
    c_out = coef_ref[1:2, :]
    a_sc = coef_ref[2:3, :]
    c_sc = coef_ref[3:4, :]
    y = zsc_ref[...] * a_sc + c_sc + zout_ref[...] * a_out + c_out
    out_ref[...] = jnp.where(y >= 0, y, 0.2 * y)


def kernel(xyz, feature, ori_relative_feature, neighbors_idx,
           W1, b1, g1, be1, W_attn, W_out, b_out, g_out, be_out,
           W_sc, b_sc, g_sc, be_sc):
    feat2 = feature.reshape(P, CIN)
    orelT = jnp.transpose(ori_relative_feature, (0, 3, 2, 1)).reshape(B, 10, K * N)
    offs = (jnp.arange(B, dtype=jnp.int32) * N)[:, None, None]
    nrk = MAIN_NP * K
    nb = R // nrk
    idx2d = ((neighbors_idx + offs).reshape(nb, 4, NT, K)
             .transpose(0, 3, 2, 1).reshape(R // 128, 128))

    fgp = _sc_gather(feat2, idx2d).reshape(RP, 128)

    nseg = N // MAIN_NP
    ncs = nseg // (A_COLS // MAIN_NP)
    yp4, sums1 = pl.pallas_call(
        _passa_body,
        grid=(B, K, ncs),
        in_specs=[
            pl.BlockSpec((1, 10, A_COLS), lambda b, k, c: (b, 0, k * ncs + c)),
            pl.BlockSpec((10, CREL), lambda b, k, c: (0, 0)),
            pl.BlockSpec((1, CREL), lambda b, k, c: (0, 0)),
        ],
        out_specs=[
            pl.BlockSpec((1, A_COLS // MAIN_NP, NT, 128),
                         lambda b, k, c: (k, b * ncs + c, 0, 0)),
            pl.BlockSpec((8, CREL), lambda b, k, c: (0, 0)),
        ],
        out_shape=[
            jax.ShapeDtypeStruct((K, nb, NT, 128), jnp.float32),
            jax.ShapeDtypeStruct((8, CREL), jnp.float32),
        ],
    )(orelT, W1, b1.reshape(1, CREL))

    m1 = sums1[0] / R
    v1 = sums1[1] / R - m1 * m1
    a1 = g1 * lax.rsqrt(v1 + EPS)
    c1 = be1 - m1 * a1
    a1t = jnp.tile(a1, 4).reshape(1, 128)
    c1t = jnp.tile(c1, 4).reshape(1, 128)

    npk = MAIN_NP * K // 4
    z_out, z_sc, sums2 = pl.pallas_call(
        _main_body,
        grid=(P // MAIN_NP,),
        in_specs=[
            pl.BlockSpec((npk, 128), lambda i: (i, 0)),
            pl.BlockSpec((K, 1, NT, 128), lambda i: (0, i, 0, 0)),
            pl.BlockSpec((MAIN_NP, CIN), lambda i: (i, 0)),
            pl.BlockSpec((1, 128), lambda i: (0, 0)),
            pl.BlockSpec((1, 128), lambda i: (0, 0)),
            pl.BlockSpec((CIN + CREL, CIN + CREL), lambda i: (0, 0)),
            pl.BlockSpec((CIN + CREL, COUT), lambda i: (0, 0)),
            pl.BlockSpec((1, COUT), lambda i: (0, 0)),
            pl.BlockSpec((CIN, COUT), lambda i: (0, 0)),
            pl.BlockSpec((1, COUT), lambda i: (0, 0)),
        ],
        out_specs=[
            pl.BlockSpec((MAIN_NP, COUT), lambda i: (i, 0)),
            pl.BlockSpec((MAIN_NP, COUT), lambda i: (i, 0)),
            pl.BlockSpec((8, COUT), lambda i: (0, 0)),
        ],
        out_shape=[
            jax.ShapeDtypeStruct((P, COUT), jnp.float32),
            jax.ShapeDtypeStruct((P, COUT), jnp.float32),
            jax.ShapeDtypeStruct((8, COUT), jnp.float32),
        ],
    )(fgp, yp4, feat2, a1t, c1t, W_attn, W_out, b_out.reshape(1, COUT),
      W_sc, b_sc.reshape(1, COUT))

    mo = sums2[0] / P
    vo = sums2[1] / P - mo * mo
    ao = g_out * lax.rsqrt(vo + EPS)
    co = be_out - mo * ao
    ms = sums2[2] / P
    vs = sums2[3] / P - ms * ms
    asc = g_sc * lax.rsqrt(vs + EPS)
    csc = be_sc - ms * asc
    coef = jnp.concatenate(
        [jnp.stack([ao, co, asc, csc]), jnp.zeros((4, COUT), jnp.float32)], axis=0)

    out = pl.pallas_call(
        _final_body,
        grid=(P // FIN_NP,),
        in_specs=[
            pl.BlockSpec((FIN_NP, COUT), lambda i: (i, 0)),
            pl.BlockSpec((FIN_NP, COUT), lambda i: (i, 0)),
            pl.BlockSpec((8, COUT), lambda i: (0, 0)),
        ],
        out_specs=pl.BlockSpec((FIN_NP, COUT), lambda i: (i, 0)),
        out_shape=jax.ShapeDtypeStruct((P, COUT), jnp.float32),
    )(z_out, z_sc, coef)

    return (xyz, out.reshape(B, N, COUT), ori_relative_feature, neighbors_idx)
